# Initial kernel scaffold; baseline (speedup 1.0000x reference)
import jax
import jax.numpy as jnp
from jax.experimental import pallas as pl


def kernel(x, embed_low, project_up):
    raise NotImplementedError("write your pallas kernel here")


# speedup vs baseline: 2.2977x; 2.2977x over previous
import functools

import jax
import jax.numpy as jnp
from jax import lax
from jax.experimental import pallas as pl
from jax.experimental.pallas import tpu as pltpu
from jax.experimental.pallas import tpu_sc as plsc

RANK = 128
D_MODEL = 1024
G = 128


def _sc_gather(table, idx3d, n_rows):
    info = plsc.get_sparse_core_info()
    nw = info.num_cores * info.num_subcores
    per_w = n_rows // nw
    n_g = per_w // G
    mesh = plsc.VectorSubcoreMesh(core_axis_name="c", subcore_axis_name="s")

    @functools.partial(
        pl.kernel,
        mesh=mesh,
        out_type=jax.ShapeDtypeStruct((n_rows, RANK), jnp.float32),
        scratch_types=[
            pltpu.VMEM((n_g, G), jnp.int32),
            pltpu.VMEM((G, RANK), jnp.float32),
            pltpu.SemaphoreType.DMA,
        ],
    )
    def k(table_hbm, idx_hbm, out_hbm, idx_v, rows_v, sem):
        wid = lax.axis_index("s") * info.num_cores + lax.axis_index("c")
        pltpu.sync_copy(idx_hbm.at[wid], idx_v)
        row_base = wid * per_w

        def body(j, carry):
            pltpu.async_copy(table_hbm.at[idx_v.at[j]], rows_v, sem).wait()
            pltpu.sync_copy(rows_v, out_hbm.at[pl.ds(row_base + j * G, G)])
            return carry

        lax.fori_loop(0, n_g, body, 0)

    return k(table, idx3d)


def _tc_project(emb, proj):
    m = emb.shape[0]
    bm = 1024
    body = lambda e_ref, p_ref, o_ref: o_ref.__setitem__(
        ..., jnp.dot(e_ref[...], p_ref[...], preferred_element_type=jnp.float32))
    return pl.pallas_call(
        body,
        grid=(m // bm,),
        in_specs=[
            pl.BlockSpec((bm, RANK), lambda i: (i, 0)),
            pl.BlockSpec((RANK, D_MODEL), lambda i: (0, 0)),
        ],
        out_specs=pl.BlockSpec((bm, D_MODEL), lambda i: (i, 0)),
        out_shape=jax.ShapeDtypeStruct((m, D_MODEL), jnp.float32),
    )(emb, proj)


def kernel(x, embed_low, project_up):
    b, l = x.shape
    n_rows = b * l
    idx3d = x.reshape(32, n_rows // (32 * G), G).astype(jnp.int32)
    emb = _sc_gather(embed_low, idx3d, n_rows)
    out = _tc_project(emb, project_up)
    return out.reshape(b, l, D_MODEL)


# speedup vs baseline: 3.4787x; 1.5140x over previous
import functools

import jax
import jax.numpy as jnp
from jax import lax
from jax.experimental import pallas as pl
from jax.experimental.pallas import tpu as pltpu
from jax.experimental.pallas import tpu_sc as plsc

RANK = 128
D_MODEL = 1024
G = 128


def _sc_gather(table, idx3d, n_rows):
    info = plsc.get_sparse_core_info()
    nw = info.num_cores * info.num_subcores
    per_w = n_rows // nw
    n_g = per_w // G
    mesh = plsc.VectorSubcoreMesh(core_axis_name="c", subcore_axis_name="s")

    @functools.partial(
        pl.kernel,
        mesh=mesh,
        out_type=jax.ShapeDtypeStruct((n_rows, RANK), jnp.float32),
        scratch_types=[
            pltpu.VMEM((n_g, G), jnp.int32),
            pltpu.VMEM((G, RANK), jnp.float32),
            pltpu.SemaphoreType.DMA,
        ],
    )
    def k(table_hbm, idx_hbm, out_hbm, idx_v, rows_v, sem):
        wid = lax.axis_index("s") * info.num_cores + lax.axis_index("c")
        pltpu.sync_copy(idx_hbm.at[wid], idx_v)
        row_base = wid * per_w

        def body(j, carry):
            pltpu.async_copy(table_hbm.at[idx_v.at[j]], rows_v, sem).wait()
            pltpu.sync_copy(rows_v, out_hbm.at[pl.ds(row_base + j * G, G)])
            return carry

        lax.fori_loop(0, n_g, body, 0)

    return k(table, idx3d)


def _tc_project(emb, proj, b, l):
    bb = 32

    def body(e_ref, p_ref, o_ref):
        p = p_ref[...]
        for i in range(bb):
            o_ref[i] = jnp.dot(e_ref[pl.ds(i * l, l), :], p,
                               preferred_element_type=jnp.float32)

    return pl.pallas_call(
        body,
        grid=(b // bb,),
        in_specs=[
            pl.BlockSpec((bb * l, RANK), lambda i: (i, 0)),
            pl.BlockSpec((RANK, D_MODEL), lambda i: (0, 0)),
        ],
        out_specs=pl.BlockSpec((bb, l, D_MODEL), lambda i: (i, 0, 0)),
        out_shape=jax.ShapeDtypeStruct((b, l, D_MODEL), jnp.float32),
    )(emb, proj)


def kernel(x, embed_low, project_up):
    b, l = x.shape
    n_rows = b * l
    idx3d = x.reshape(32, n_rows // (32 * G), G).astype(jnp.int32)
    emb = _sc_gather(embed_low, idx3d, n_rows)
    return _tc_project(emb, project_up, b, l)
